# baseline (speedup 1.0000x reference)
import functools

import jax
import jax.numpy as jnp
from jax import lax
from jax.experimental import pallas as pl
from jax.experimental.pallas import tpu as pltpu
from jax.experimental.pallas import tpu_sc as plsc

B, N, K, P, C = 4, 16384, 512, 64, 256
RADIUS = 0.1
POS_HID = 64
ATTN_HID = C * 4 // 4
ATTN_OUT = C // 4
M = B * K

KB = 64
NC = 2048


def _bq_body(qe_ref, xe_ref, idx_ref, raw_ref, carry_ref):
    nc = pl.program_id(2)

    @pl.when(nc == 0)
    def _init():
        raw_ref[...] = jnp.zeros_like(raw_ref)
        carry_ref[...] = jnp.zeros_like(carry_ref)

    qe = qe_ref[0]
    xe = xe_ref[0]
    qb = qe.astype(jnp.bfloat16)
    xb = xe.astype(jnp.bfloat16)
    mm = jax.lax.dot_general(qb, xb, (((1,), (0,)), ((), ())),
                             preferred_element_type=jnp.float32)
    d2 = (qe[:, 3:4] + xe[4:5, :]) - 2.0 * mm
    mask = (d2 <= RADIUS ** 2).astype(jnp.int32)
    c = mask
    sh = 1
    while sh < NC:
        z = jnp.zeros((KB, sh), jnp.int32)
        c = c + jnp.concatenate([z, c[:, :NC - sh]], axis=1)
        sh *= 2
    c = c + carry_ref[:, :1]
    cnt = c[:, NC - 1:NC]
    carry_ref[:, :1] = cnt
    cols = [jnp.sum((c <= p).astype(jnp.int32), axis=1, keepdims=True)
            for p in range(P)]
    raw_ref[...] += jnp.concatenate(cols, axis=1)

    @pl.when(nc == pl.num_programs(2) - 1)
    def _fin():
        raw = raw_ref[...]
        pos = jax.lax.broadcasted_iota(jnp.int32, (KB, P), 1)
        r = jnp.where(pos < cnt, raw, raw[:, :1])
        r = jnp.where(cnt == 0, 0, r)
        idx_ref[0] = r + pl.program_id(0) * N


def _ball_query(qe, xeT):
    return pl.pallas_call(
        _bq_body,
        grid=(B, K // KB, N // NC),
        in_specs=[
            pl.BlockSpec((1, KB, 8), lambda b, kb, nc: (b, kb, 0)),
            pl.BlockSpec((1, 8, NC), lambda b, kb, nc: (b, 0, nc)),
        ],
        out_specs=pl.BlockSpec((1, KB, P), lambda b, kb, nc: (b, kb, 0)),
        out_shape=jax.ShapeDtypeStruct((B, K, P), jnp.int32),
        scratch_shapes=[
            pltpu.VMEM((KB, P), jnp.int32),
            pltpu.VMEM((KB, 128), jnp.int32),
        ],
    )(qe, xeT)


GCH = 128
ROWS = M * P


SC_CORES = 2
SC_SUBCORES = 16


@functools.cache
def _make_gather():
    nw = SC_CORES * SC_SUBCORES
    per_w = ROWS // nw
    n_ch = per_w // GCH
    mesh = plsc.VectorSubcoreMesh(core_axis_name="c", subcore_axis_name="s",
                                  num_cores=SC_CORES,
                                  num_subcores=SC_SUBCORES)

    @functools.partial(
        pl.kernel, mesh=mesh,
        out_type=[jax.ShapeDtypeStruct((ROWS, C), jnp.float32),
                  jax.ShapeDtypeStruct((ROWS, 128), jnp.float32)],
        scratch_types=[
            pltpu.VMEM((GCH,), jnp.int32),
            pltpu.VMEM((GCH, C), jnp.float32),
            pltpu.VMEM((GCH, 128), jnp.float32),
            pltpu.SemaphoreType.DMA,
        ],
    )
    def gather(ft_hbm, xyzp_hbm, gidx_hbm, outf_hbm, outx_hbm,
               idx_v, rowf_v, rowx_v, sem):
        wid = lax.axis_index("s") * SC_CORES + lax.axis_index("c")
        base = wid * per_w

        def body(j, _):
            off = base + j * GCH
            pltpu.sync_copy(gidx_hbm.at[pl.ds(off, GCH)], idx_v)
            pltpu.async_copy(ft_hbm.at[idx_v], rowf_v, sem).wait()
            pltpu.sync_copy(rowf_v, outf_hbm.at[pl.ds(off, GCH)])
            pltpu.async_copy(xyzp_hbm.at[idx_v], rowx_v, sem).wait()
            pltpu.sync_copy(rowx_v, outx_hbm.at[pl.ds(off, GCH)])
            return 0

        lax.fori_loop(0, n_ch, body, 0)

    return gather


def _gather_sc(ft, xyzp, gidx):
    return _make_gather()(ft, xyzp, gidx)


G = 8


def _attn_body(nf_ref, npx_ref, pf_ref, nx_ref,
               wq_ref, wk_ref, wv_ref, pw1_ref, pb1_ref, pw2_ref, pb2_ref,
               aw1_ref, ab1_ref, aw2_ref, ab2_ref, z_ref):
    R = G * P
    bf = jnp.bfloat16
    f32 = jnp.float32
    dot = lambda a, b: jnp.dot(a.astype(bf), b.astype(bf),
                               preferred_element_type=f32)
    nf2 = nf_ref[...].reshape(R, C)
    kk = dot(nf2, wk_ref[...])
    vv = dot(nf2, wv_ref[...])
    q = dot(pf_ref[...], wq_ref[...])
    rel = nx_ref[...][:, None, :] - npx_ref[...]
    pos1 = jax.nn.relu(dot(rel.reshape(R, 128), pw1_ref[...]) + pb1_ref[...])
    pos = dot(pos1, pw2_ref[...]) + pb2_ref[...]
    t = q[:, None, :] - kk.reshape(G, P, C) + pos.reshape(G, P, C)
    s1 = jax.nn.relu(dot(t.reshape(R, C), aw1_ref[...]) + ab1_ref[...])
    sim = dot(s1, aw2_ref[...]) + ab2_ref[...]
    sim3 = sim.reshape(G, P, ATTN_OUT)
    mx = jnp.max(sim3, axis=1, keepdims=True)
    e = jnp.exp(sim3 - mx)
    a = e / jnp.sum(e, axis=1, keepdims=True)
    ri = jax.lax.broadcasted_iota(jnp.int32, (ATTN_OUT, C), 0)
    cj = jax.lax.broadcasted_iota(jnp.int32, (ATTN_OUT, C), 1)
    expand = (cj // (C // ATTN_OUT) == ri).astype(jnp.float32)
    aexp = jnp.dot(a.reshape(R, ATTN_OUT), expand,
                   preferred_element_type=jnp.float32)
    vpos = vv + pos
    agg = jnp.sum(aexp.reshape(G, P, C) * vpos.reshape(G, P, C), axis=1)
    z_ref[...] = pf_ref[...] + agg


def _attention(nf, npx, pf, nx, Wq, Wk, Wv, pw1, pb1, pw2, pb2,
               aw1, ab1, aw2, ab2):
    full = lambda m: (0, 0)
    return pl.pallas_call(
        _attn_body,
        grid=(M // G,),
        in_specs=[
            pl.BlockSpec((G, P, C), lambda m: (m, 0, 0)),
            pl.BlockSpec((G, P, 128), lambda m: (m, 0, 0)),
            pl.BlockSpec((G, C), lambda m: (m, 0)),
            pl.BlockSpec((G, 128), lambda m: (m, 0)),
            pl.BlockSpec((C, C), full),
            pl.BlockSpec((C, C), full),
            pl.BlockSpec((C, C), full),
            pl.BlockSpec((128, POS_HID), full),
            pl.BlockSpec((1, POS_HID), full),
            pl.BlockSpec((POS_HID, C), full),
            pl.BlockSpec((1, C), full),
            pl.BlockSpec((C, ATTN_HID), full),
            pl.BlockSpec((1, ATTN_HID), full),
            pl.BlockSpec((ATTN_HID, ATTN_OUT), full),
            pl.BlockSpec((1, ATTN_OUT), full),
        ],
        out_specs=pl.BlockSpec((G, C), lambda m: (m, 0)),
        out_shape=jax.ShapeDtypeStruct((M, C), jnp.float32),
    )(nf, npx, pf, nx, Wq, Wk, Wv, pw1, pb1, pw2, pb2, aw1, ab1, aw2, ab2)


def _head_body(zp_ref, sw_ref, g1_ref, b1_ref, pw_ref, g2_ref, b2_ref,
               zi_ref, zb_ref):
    bf = jnp.bfloat16
    h1 = jnp.dot(zp_ref[...].astype(bf), sw_ref[...].astype(bf),
                 preferred_element_type=jnp.float32)
    m1 = jnp.mean(h1, axis=0, keepdims=True)
    v1 = jnp.mean((h1 - m1) ** 2, axis=0, keepdims=True)
    zb = jax.nn.relu(g1_ref[...] * (h1 - m1) / jnp.sqrt(v1 + 1e-3)
                     + b1_ref[...])
    h2 = jnp.dot(zb.astype(bf), pw_ref[...].astype(bf),
                 preferred_element_type=jnp.float32)
    m2 = jnp.mean(h2, axis=0, keepdims=True)
    v2 = jnp.mean((h2 - m2) ** 2, axis=0, keepdims=True)
    zi = g2_ref[...] * (h2 - m2) / jnp.sqrt(v2 + 1e-3) + b2_ref[...]
    zb_ref[...] = zb
    zi_ref[...] = zi


def _heads(zp, share_w, bn1_g, bn1_b, patch_w, bn2_g, bn2_b):
    return pl.pallas_call(
        _head_body,
        out_shape=[jax.ShapeDtypeStruct((M, 256), jnp.float32),
                   jax.ShapeDtypeStruct((M, 512), jnp.float32)],
    )(zp, share_w, bn1_g.reshape(1, -1), bn1_b.reshape(1, -1),
      patch_w, bn2_g.reshape(1, -1), bn2_b.reshape(1, -1))


def kernel(xyz, new_xyz, features, point_feat, Wq, Wk, Wv, pos_w1, pos_b1,
           pos_w2, pos_b2, attn_w1, attn_b1, attn_w2, attn_b2, share_w,
           bn1_g, bn1_b, patch_w, bn2_g, bn2_b):
    qn = jnp.sum(new_xyz ** 2, axis=-1, keepdims=True)
    qe = jnp.concatenate([new_xyz, qn, jnp.zeros((B, K, 4))], axis=-1)
    xn = jnp.sum(xyz ** 2, axis=-1, keepdims=True)
    xe = jnp.concatenate([xyz, jnp.zeros((B, N, 1)), xn,
                          jnp.zeros((B, N, 3))], axis=-1)
    xeT = jnp.transpose(xe, (0, 2, 1))

    gidx = _ball_query(qe, xeT)

    ft = jnp.transpose(features, (0, 2, 1)).reshape(B * N, C)
    xyzp = jnp.pad(xyz.reshape(B * N, 3), ((0, 0), (0, 125)))
    nf_rows, nx_rows = _gather_sc(ft, xyzp, gidx.reshape(ROWS))

    nf = nf_rows.reshape(M, P, C)
    npx = nx_rows.reshape(M, P, 128)
    pf = point_feat.reshape(M, C)
    nxp = jnp.pad(new_xyz.reshape(M, 3), ((0, 0), (0, 125)))

    pw1 = jnp.pad(pos_w1, ((0, 125), (0, 0)))
    z = _attention(nf, npx, pf, nxp, Wq, Wk, Wv, pw1,
                   pos_b1.reshape(1, -1), pos_w2, pos_b2.reshape(1, -1),
                   attn_w1, attn_b1.reshape(1, -1), attn_w2,
                   attn_b2.reshape(1, -1))

    zp = jnp.transpose(z.reshape(B // 2, 2, K, C), (0, 2, 1, 3)).reshape(M, C)
    zi, zb = _heads(zp, share_w, bn1_g, bn1_b, patch_w, bn2_g, bn2_b)
    return (zi, zb)


# speedup vs baseline: 1.0877x; 1.0877x over previous
import functools

import jax
import jax.numpy as jnp
from jax import lax
from jax.experimental import pallas as pl
from jax.experimental.pallas import tpu as pltpu
from jax.experimental.pallas import tpu_sc as plsc

B, N, K, P, C = 4, 16384, 512, 64, 256
RADIUS = 0.1
POS_HID = 64
ATTN_HID = C * 4 // 4
ATTN_OUT = C // 4
M = B * K

KB = 64
NC = 2048


def _bq_body(qe_ref, xe_ref, idx_ref, raw_ref, carry_ref):
    nc = pl.program_id(2)

    @pl.when(nc == 0)
    def _init():
        raw_ref[...] = jnp.zeros_like(raw_ref)
        carry_ref[...] = jnp.zeros_like(carry_ref)

    qe = qe_ref[0]
    xe = xe_ref[0]
    qb = qe.astype(jnp.bfloat16)
    xb = xe.astype(jnp.bfloat16)
    mm = jax.lax.dot_general(qb, xb, (((1,), (0,)), ((), ())),
                             preferred_element_type=jnp.float32)
    d2 = (qe[:, 3:4] + xe[4:5, :]) - 2.0 * mm
    mask = (d2 <= RADIUS ** 2).astype(jnp.int32)
    c = mask
    sh = 1
    while sh < NC:
        z = jnp.zeros((KB, sh), jnp.int32)
        c = c + jnp.concatenate([z, c[:, :NC - sh]], axis=1)
        sh *= 2
    c = c + carry_ref[:, :1]
    cnt = c[:, NC - 1:NC]
    carry_ref[:, :1] = cnt
    cols = [jnp.sum((c <= p).astype(jnp.int32), axis=1, keepdims=True)
            for p in range(P)]
    raw_ref[...] += jnp.concatenate(cols, axis=1)

    @pl.when(nc == pl.num_programs(2) - 1)
    def _fin():
        raw = raw_ref[...]
        pos = jax.lax.broadcasted_iota(jnp.int32, (KB, P), 1)
        r = jnp.where(pos < cnt, raw, raw[:, :1])
        r = jnp.where(cnt == 0, 0, r)
        idx_ref[0] = r + pl.program_id(0) * N


def _ball_query(qe, xeT):
    return pl.pallas_call(
        _bq_body,
        grid=(B, K // KB, N // NC),
        in_specs=[
            pl.BlockSpec((1, KB, 8), lambda b, kb, nc: (b, kb, 0)),
            pl.BlockSpec((1, 8, NC), lambda b, kb, nc: (b, 0, nc)),
        ],
        out_specs=pl.BlockSpec((1, KB, P), lambda b, kb, nc: (b, kb, 0)),
        out_shape=jax.ShapeDtypeStruct((B, K, P), jnp.int32),
        scratch_shapes=[
            pltpu.VMEM((KB, P), jnp.int32),
            pltpu.VMEM((KB, 128), jnp.int32),
        ],
    )(qe, xeT)


GCH = 128
ROWS = M * P


SC_CORES = 2
SC_SUBCORES = 16


@functools.cache
def _make_gather():
    nw = SC_CORES * SC_SUBCORES
    per_w = ROWS // nw
    n_ch = per_w // GCH
    mesh = plsc.VectorSubcoreMesh(core_axis_name="c", subcore_axis_name="s",
                                  num_cores=SC_CORES,
                                  num_subcores=SC_SUBCORES)

    @functools.partial(
        pl.kernel, mesh=mesh,
        out_type=[jax.ShapeDtypeStruct((ROWS, C), jnp.float32),
                  jax.ShapeDtypeStruct((ROWS, 128), jnp.float32)],
        scratch_types=[
            pltpu.VMEM((GCH,), jnp.int32),
            pltpu.VMEM((GCH, C), jnp.float32),
            pltpu.VMEM((GCH, 128), jnp.float32),
            pltpu.SemaphoreType.DMA,
            pltpu.SemaphoreType.DMA,
        ],
    )
    def gather(ft_hbm, xyzp_hbm, gidx_hbm, outf_hbm, outx_hbm,
               idx_v, rowf_v, rowx_v, semf, semx):
        wid = lax.axis_index("s") * SC_CORES + lax.axis_index("c")
        base = wid * per_w

        def body(j, _):
            off = base + j * GCH
            pltpu.sync_copy(gidx_hbm.at[pl.ds(off, GCH)], idx_v)
            cf = pltpu.async_copy(ft_hbm.at[idx_v], rowf_v, semf)
            cx = pltpu.async_copy(xyzp_hbm.at[idx_v], rowx_v, semx)
            cf.wait()
            pltpu.sync_copy(rowf_v, outf_hbm.at[pl.ds(off, GCH)])
            cx.wait()
            pltpu.sync_copy(rowx_v, outx_hbm.at[pl.ds(off, GCH)])
            return 0

        lax.fori_loop(0, n_ch, body, 0)

    return gather


def _gather_sc(ft, xyzp, gidx):
    return _make_gather()(ft, xyzp, gidx)


G = 16


def _attn_body(nf_ref, npx_ref, pf_ref, nx_ref,
               wq_ref, wk_ref, wv_ref, pw1_ref, pb1_ref, pw2_ref, pb2_ref,
               aw1_ref, ab1_ref, aw2_ref, ab2_ref, z_ref):
    R = G * P
    bf = jnp.bfloat16
    f32 = jnp.float32
    dot = lambda a, b: jnp.dot(a.astype(bf), b.astype(bf),
                               preferred_element_type=f32)
    nf2 = nf_ref[...].reshape(R, C)
    kk = dot(nf2, wk_ref[...])
    vv = dot(nf2, wv_ref[...])
    q = dot(pf_ref[...], wq_ref[...])
    rel = nx_ref[...][:, None, :] - npx_ref[...]
    pos1 = jax.nn.relu(dot(rel.reshape(R, 128), pw1_ref[...]) + pb1_ref[...])
    pos = dot(pos1, pw2_ref[...]) + pb2_ref[...]
    t = q[:, None, :] - kk.reshape(G, P, C) + pos.reshape(G, P, C)
    s1 = jax.nn.relu(dot(t.reshape(R, C), aw1_ref[...]) + ab1_ref[...])
    sim = dot(s1, aw2_ref[...]) + ab2_ref[...]
    sim3 = sim.reshape(G, P, ATTN_OUT)
    mx = jnp.max(sim3, axis=1, keepdims=True)
    e = jnp.exp(sim3 - mx)
    a = e / jnp.sum(e, axis=1, keepdims=True)
    ri = jax.lax.broadcasted_iota(jnp.int32, (ATTN_OUT, C), 0)
    cj = jax.lax.broadcasted_iota(jnp.int32, (ATTN_OUT, C), 1)
    expand = (cj // (C // ATTN_OUT) == ri).astype(jnp.float32)
    aexp = dot(a.reshape(R, ATTN_OUT), expand)
    vpos = vv + pos
    agg = jnp.sum(aexp.reshape(G, P, C) * vpos.reshape(G, P, C), axis=1)
    z_ref[...] = pf_ref[...] + agg


def _attention(nf, npx, pf, nx, Wq, Wk, Wv, pw1, pb1, pw2, pb2,
               aw1, ab1, aw2, ab2):
    full = lambda m: (0, 0)
    return pl.pallas_call(
        _attn_body,
        grid=(M // G,),
        in_specs=[
            pl.BlockSpec((G, P, C), lambda m: (m, 0, 0)),
            pl.BlockSpec((G, P, 128), lambda m: (m, 0, 0)),
            pl.BlockSpec((G, C), lambda m: (m, 0)),
            pl.BlockSpec((G, 128), lambda m: (m, 0)),
            pl.BlockSpec((C, C), full),
            pl.BlockSpec((C, C), full),
            pl.BlockSpec((C, C), full),
            pl.BlockSpec((128, POS_HID), full),
            pl.BlockSpec((1, POS_HID), full),
            pl.BlockSpec((POS_HID, C), full),
            pl.BlockSpec((1, C), full),
            pl.BlockSpec((C, ATTN_HID), full),
            pl.BlockSpec((1, ATTN_HID), full),
            pl.BlockSpec((ATTN_HID, ATTN_OUT), full),
            pl.BlockSpec((1, ATTN_OUT), full),
        ],
        out_specs=pl.BlockSpec((G, C), lambda m: (m, 0)),
        out_shape=jax.ShapeDtypeStruct((M, C), jnp.float32),
    )(nf, npx, pf, nx, Wq, Wk, Wv, pw1, pb1, pw2, pb2, aw1, ab1, aw2, ab2)


def _head_body(zp_ref, sw_ref, g1_ref, b1_ref, pw_ref, g2_ref, b2_ref,
               zi_ref, zb_ref):
    bf = jnp.bfloat16
    h1 = jnp.dot(zp_ref[...].astype(bf), sw_ref[...].astype(bf),
                 preferred_element_type=jnp.float32)
    m1 = jnp.mean(h1, axis=0, keepdims=True)
    v1 = jnp.mean((h1 - m1) ** 2, axis=0, keepdims=True)
    zb = jax.nn.relu(g1_ref[...] * (h1 - m1) / jnp.sqrt(v1 + 1e-3)
                     + b1_ref[...])
    h2 = jnp.dot(zb.astype(bf), pw_ref[...].astype(bf),
                 preferred_element_type=jnp.float32)
    m2 = jnp.mean(h2, axis=0, keepdims=True)
    v2 = jnp.mean((h2 - m2) ** 2, axis=0, keepdims=True)
    zi = g2_ref[...] * (h2 - m2) / jnp.sqrt(v2 + 1e-3) + b2_ref[...]
    zb_ref[...] = zb
    zi_ref[...] = zi


def _heads(zp, share_w, bn1_g, bn1_b, patch_w, bn2_g, bn2_b):
    return pl.pallas_call(
        _head_body,
        out_shape=[jax.ShapeDtypeStruct((M, 256), jnp.float32),
                   jax.ShapeDtypeStruct((M, 512), jnp.float32)],
    )(zp, share_w, bn1_g.reshape(1, -1), bn1_b.reshape(1, -1),
      patch_w, bn2_g.reshape(1, -1), bn2_b.reshape(1, -1))


def kernel(xyz, new_xyz, features, point_feat, Wq, Wk, Wv, pos_w1, pos_b1,
           pos_w2, pos_b2, attn_w1, attn_b1, attn_w2, attn_b2, share_w,
           bn1_g, bn1_b, patch_w, bn2_g, bn2_b):
    qn = jnp.sum(new_xyz ** 2, axis=-1, keepdims=True)
    qe = jnp.concatenate([new_xyz, qn, jnp.zeros((B, K, 4))], axis=-1)
    xn = jnp.sum(xyz ** 2, axis=-1, keepdims=True)
    xe = jnp.concatenate([xyz, jnp.zeros((B, N, 1)), xn,
                          jnp.zeros((B, N, 3))], axis=-1)
    xeT = jnp.transpose(xe, (0, 2, 1))

    gidx = _ball_query(qe, xeT)

    ft = jnp.transpose(features, (0, 2, 1)).reshape(B * N, C)
    xyzp = jnp.pad(xyz.reshape(B * N, 3), ((0, 0), (0, 125)))
    nf_rows, nx_rows = _gather_sc(ft, xyzp, gidx.reshape(ROWS))

    nf = nf_rows.reshape(M, P, C)
    npx = nx_rows.reshape(M, P, 128)
    pf = point_feat.reshape(M, C)
    nxp = jnp.pad(new_xyz.reshape(M, 3), ((0, 0), (0, 125)))

    pw1 = jnp.pad(pos_w1, ((0, 125), (0, 0)))
    z = _attention(nf, npx, pf, nxp, Wq, Wk, Wv, pw1,
                   pos_b1.reshape(1, -1), pos_w2, pos_b2.reshape(1, -1),
                   attn_w1, attn_b1.reshape(1, -1), attn_w2,
                   attn_b2.reshape(1, -1))

    zp = jnp.transpose(z.reshape(B // 2, 2, K, C), (0, 2, 1, 3)).reshape(M, C)
    zi, zb = _heads(zp, share_w, bn1_g, bn1_b, patch_w, bn2_g, bn2_b)
    return (zi, zb)


# speedup vs baseline: 1.1233x; 1.0328x over previous
import functools

import jax
import jax.numpy as jnp
from jax import lax
from jax.experimental import pallas as pl
from jax.experimental.pallas import tpu as pltpu
from jax.experimental.pallas import tpu_sc as plsc

B, N, K, P, C = 4, 16384, 512, 64, 256
RADIUS = 0.1
POS_HID = 64
ATTN_HID = C * 4 // 4
ATTN_OUT = C // 4
M = B * K

KB = 128
NC = 2048


def _bq_body(qe_ref, xe_ref, idx_ref, raw_ref, carry_ref):
    nc = pl.program_id(2)

    @pl.when(nc == 0)
    def _init():
        raw_ref[...] = jnp.zeros_like(raw_ref)
        carry_ref[...] = jnp.zeros_like(carry_ref)

    qe = qe_ref[0]
    xe = xe_ref[0]
    qb = qe.astype(jnp.bfloat16)
    xb = xe.astype(jnp.bfloat16)
    mm = jax.lax.dot_general(qb, xb, (((1,), (0,)), ((), ())),
                             preferred_element_type=jnp.float32)
    d2 = (qe[:, 3:4] + xe[4:5, :]) - 2.0 * mm
    mask = (d2 <= RADIUS ** 2).astype(jnp.int32)
    c = mask
    sh = 1
    while sh < NC:
        z = jnp.zeros((KB, sh), jnp.int32)
        c = c + jnp.concatenate([z, c[:, :NC - sh]], axis=1)
        sh *= 2
    c = c + carry_ref[:, :1]
    cnt = c[:, NC - 1:NC]
    carry_ref[:, :1] = cnt
    cols = [jnp.sum((c <= p).astype(jnp.int32), axis=1, keepdims=True)
            for p in range(P)]
    raw_ref[...] += jnp.concatenate(cols, axis=1)

    @pl.when(nc == pl.num_programs(2) - 1)
    def _fin():
        raw = raw_ref[...]
        pos = jax.lax.broadcasted_iota(jnp.int32, (KB, P), 1)
        r = jnp.where(pos < cnt, raw, raw[:, :1])
        r = jnp.where(cnt == 0, 0, r)
        idx_ref[0] = r + pl.program_id(0) * N


def _ball_query(qe, xeT):
    return pl.pallas_call(
        _bq_body,
        grid=(B, K // KB, N // NC),
        in_specs=[
            pl.BlockSpec((1, KB, 8), lambda b, kb, nc: (b, kb, 0)),
            pl.BlockSpec((1, 8, NC), lambda b, kb, nc: (b, 0, nc)),
        ],
        out_specs=pl.BlockSpec((1, KB, P), lambda b, kb, nc: (b, kb, 0)),
        out_shape=jax.ShapeDtypeStruct((B, K, P), jnp.int32),
        scratch_shapes=[
            pltpu.VMEM((KB, P), jnp.int32),
            pltpu.VMEM((KB, 128), jnp.int32),
        ],
    )(qe, xeT)


GCH = 128
ROWS = M * P


SC_CORES = 2
SC_SUBCORES = 16


@functools.cache
def _make_gather():
    nw = SC_CORES * SC_SUBCORES
    per_w = ROWS // nw
    n_ch = per_w // GCH
    mesh = plsc.VectorSubcoreMesh(core_axis_name="c", subcore_axis_name="s",
                                  num_cores=SC_CORES,
                                  num_subcores=SC_SUBCORES)

    @functools.partial(
        pl.kernel, mesh=mesh,
        out_type=[jax.ShapeDtypeStruct((ROWS, C), jnp.float32),
                  jax.ShapeDtypeStruct((ROWS, 128), jnp.float32)],
        scratch_types=[
            pltpu.VMEM((GCH,), jnp.int32),
            pltpu.VMEM((GCH, C), jnp.float32),
            pltpu.VMEM((GCH, 128), jnp.float32),
            pltpu.SemaphoreType.DMA,
            pltpu.SemaphoreType.DMA,
        ],
    )
    def gather(ft_hbm, xyzp_hbm, gidx_hbm, outf_hbm, outx_hbm,
               idx_v, rowf_v, rowx_v, semf, semx):
        wid = lax.axis_index("s") * SC_CORES + lax.axis_index("c")
        base = wid * per_w

        def body(j, _):
            off = base + j * GCH
            pltpu.sync_copy(gidx_hbm.at[pl.ds(off, GCH)], idx_v)
            cf = pltpu.async_copy(ft_hbm.at[idx_v], rowf_v, semf)
            cx = pltpu.async_copy(xyzp_hbm.at[idx_v], rowx_v, semx)
            cf.wait()
            pltpu.sync_copy(rowf_v, outf_hbm.at[pl.ds(off, GCH)])
            cx.wait()
            pltpu.sync_copy(rowx_v, outx_hbm.at[pl.ds(off, GCH)])
            return 0

        lax.fori_loop(0, n_ch, body, 0)

    return gather


def _gather_sc(ft, xyzp, gidx):
    return _make_gather()(ft, xyzp, gidx)


G = 32


def _attn_body(nf_ref, npx_ref, pf_ref, nx_ref,
               wq_ref, wk_ref, wv_ref, pw1_ref, pb1_ref, pw2_ref, pb2_ref,
               aw1_ref, ab1_ref, aw2_ref, ab2_ref, z_ref):
    R = G * P
    bf = jnp.bfloat16
    f32 = jnp.float32
    dot = lambda a, b: jnp.dot(a.astype(bf), b.astype(bf),
                               preferred_element_type=f32)
    nf2 = nf_ref[...].reshape(R, C)
    kk = dot(nf2, wk_ref[...])
    vv = dot(nf2, wv_ref[...])
    q = dot(pf_ref[...], wq_ref[...])
    rel = nx_ref[...][:, None, :] - npx_ref[...]
    pos1 = jax.nn.relu(dot(rel.reshape(R, 128), pw1_ref[...]) + pb1_ref[...])
    pos = dot(pos1, pw2_ref[...]) + pb2_ref[...]
    t = q[:, None, :] - kk.reshape(G, P, C) + pos.reshape(G, P, C)
    s1 = jax.nn.relu(dot(t.reshape(R, C), aw1_ref[...]) + ab1_ref[...])
    sim = dot(s1, aw2_ref[...]) + ab2_ref[...]
    sim3 = sim.reshape(G, P, ATTN_OUT)
    mx = jnp.max(sim3, axis=1, keepdims=True)
    e = jnp.exp(sim3 - mx)
    a = e / jnp.sum(e, axis=1, keepdims=True)
    ri = jax.lax.broadcasted_iota(jnp.int32, (ATTN_OUT, C), 0)
    cj = jax.lax.broadcasted_iota(jnp.int32, (ATTN_OUT, C), 1)
    expand = (cj // (C // ATTN_OUT) == ri).astype(jnp.float32)
    aexp = dot(a.reshape(R, ATTN_OUT), expand)
    vpos = vv + pos
    agg = jnp.sum(aexp.reshape(G, P, C) * vpos.reshape(G, P, C), axis=1)
    z_ref[...] = pf_ref[...] + agg


def _attention(nf, npx, pf, nx, Wq, Wk, Wv, pw1, pb1, pw2, pb2,
               aw1, ab1, aw2, ab2):
    full = lambda m: (0, 0)
    return pl.pallas_call(
        _attn_body,
        grid=(M // G,),
        in_specs=[
            pl.BlockSpec((G, P, C), lambda m: (m, 0, 0)),
            pl.BlockSpec((G, P, 128), lambda m: (m, 0, 0)),
            pl.BlockSpec((G, C), lambda m: (m, 0)),
            pl.BlockSpec((G, 128), lambda m: (m, 0)),
            pl.BlockSpec((C, C), full),
            pl.BlockSpec((C, C), full),
            pl.BlockSpec((C, C), full),
            pl.BlockSpec((128, POS_HID), full),
            pl.BlockSpec((1, POS_HID), full),
            pl.BlockSpec((POS_HID, C), full),
            pl.BlockSpec((1, C), full),
            pl.BlockSpec((C, ATTN_HID), full),
            pl.BlockSpec((1, ATTN_HID), full),
            pl.BlockSpec((ATTN_HID, ATTN_OUT), full),
            pl.BlockSpec((1, ATTN_OUT), full),
        ],
        out_specs=pl.BlockSpec((G, C), lambda m: (m, 0)),
        out_shape=jax.ShapeDtypeStruct((M, C), jnp.float32),
    )(nf, npx, pf, nx, Wq, Wk, Wv, pw1, pb1, pw2, pb2, aw1, ab1, aw2, ab2)


def _head_body(zp_ref, sw_ref, g1_ref, b1_ref, pw_ref, g2_ref, b2_ref,
               zi_ref, zb_ref):
    bf = jnp.bfloat16
    h1 = jnp.dot(zp_ref[...].astype(bf), sw_ref[...].astype(bf),
                 preferred_element_type=jnp.float32)
    m1 = jnp.mean(h1, axis=0, keepdims=True)
    v1 = jnp.mean((h1 - m1) ** 2, axis=0, keepdims=True)
    zb = jax.nn.relu(g1_ref[...] * (h1 - m1) / jnp.sqrt(v1 + 1e-3)
                     + b1_ref[...])
    h2 = jnp.dot(zb.astype(bf), pw_ref[...].astype(bf),
                 preferred_element_type=jnp.float32)
    m2 = jnp.mean(h2, axis=0, keepdims=True)
    v2 = jnp.mean((h2 - m2) ** 2, axis=0, keepdims=True)
    zi = g2_ref[...] * (h2 - m2) / jnp.sqrt(v2 + 1e-3) + b2_ref[...]
    zb_ref[...] = zb
    zi_ref[...] = zi


def _heads(zp, share_w, bn1_g, bn1_b, patch_w, bn2_g, bn2_b):
    return pl.pallas_call(
        _head_body,
        out_shape=[jax.ShapeDtypeStruct((M, 256), jnp.float32),
                   jax.ShapeDtypeStruct((M, 512), jnp.float32)],
    )(zp, share_w, bn1_g.reshape(1, -1), bn1_b.reshape(1, -1),
      patch_w, bn2_g.reshape(1, -1), bn2_b.reshape(1, -1))


def kernel(xyz, new_xyz, features, point_feat, Wq, Wk, Wv, pos_w1, pos_b1,
           pos_w2, pos_b2, attn_w1, attn_b1, attn_w2, attn_b2, share_w,
           bn1_g, bn1_b, patch_w, bn2_g, bn2_b):
    qn = jnp.sum(new_xyz ** 2, axis=-1, keepdims=True)
    qe = jnp.concatenate([new_xyz, qn, jnp.zeros((B, K, 4))], axis=-1)
    xn = jnp.sum(xyz ** 2, axis=-1, keepdims=True)
    xe = jnp.concatenate([xyz, jnp.zeros((B, N, 1)), xn,
                          jnp.zeros((B, N, 3))], axis=-1)
    xeT = jnp.transpose(xe, (0, 2, 1))

    gidx = _ball_query(qe, xeT)

    ft = jnp.transpose(features, (0, 2, 1)).reshape(B * N, C)
    xyzp = jnp.pad(xyz.reshape(B * N, 3), ((0, 0), (0, 125)))
    nf_rows, nx_rows = _gather_sc(ft, xyzp, gidx.reshape(ROWS))

    nf = nf_rows.reshape(M, P, C)
    npx = nx_rows.reshape(M, P, 128)
    pf = point_feat.reshape(M, C)
    nxp = jnp.pad(new_xyz.reshape(M, 3), ((0, 0), (0, 125)))

    pw1 = jnp.pad(pos_w1, ((0, 125), (0, 0)))
    z = _attention(nf, npx, pf, nxp, Wq, Wk, Wv, pw1,
                   pos_b1.reshape(1, -1), pos_w2, pos_b2.reshape(1, -1),
                   attn_w1, attn_b1.reshape(1, -1), attn_w2,
                   attn_b2.reshape(1, -1))

    zp = jnp.transpose(z.reshape(B // 2, 2, K, C), (0, 2, 1, 3)).reshape(M, C)
    zi, zb = _heads(zp, share_w, bn1_g, bn1_b, patch_w, bn2_g, bn2_b)
    return (zi, zb)
